# baseline (speedup 1.0000x reference)
import functools

import jax
import jax.numpy as jnp
from jax import lax
from jax.experimental import pallas as pl
from jax.experimental.pallas import tpu as pltpu
from jax.experimental.pallas import tpu_sc as plsc

F = 26
FIELD = 100000
B = 16384
NC, NS, L = 2, 16, 16
NW = NC * NS
BPW = B // NW
E = BPW * F
OUTV = BPW // L
TOTAL = F * FIELD

_mesh = plsc.VectorSubcoreMesh(core_axis_name="c", subcore_axis_name="s")


@functools.partial(
    pl.kernel,
    out_type=jax.ShapeDtypeStruct((1, B), jnp.float32),
    mesh=_mesh,
    scratch_types=[
        pltpu.VMEM((E,), jnp.int32),
        pltpu.VMEM((E,), jnp.float32),
        pltpu.VMEM((BPW,), jnp.float32),
        pltpu.VMEM((L,), jnp.float32),
        pltpu.SemaphoreType.DMA,
        pltpu.SemaphoreType.DMA,
    ],
)
def _fl_kernel(xt_hbm, table_hbm, bias_hbm, out_hbm, idx_v, vals_v, acc_v,
               bias_v, semx, semg):
    wid = lax.axis_index("s") * NC + lax.axis_index("c")
    base = wid * BPW
    flat = table_hbm.at[0]

    xdescs = [
        pltpu.async_copy(
            xt_hbm.at[f, pl.ds(base, BPW)], idx_v.at[pl.ds(f * BPW, BPW)], semx
        )
        for f in range(F)
    ]
    bdesc = pltpu.async_copy(bias_hbm, bias_v.at[pl.ds(0, 1)], semx)
    gdescs = []
    for f in range(F):
        xdescs[f].wait()
        gdescs.append(
            pltpu.async_copy(
                flat.at[pl.ds(f * FIELD, FIELD)].at[idx_v.at[pl.ds(f * BPW, BPW)]],
                vals_v.at[pl.ds(f * BPW, BPW)],
                semg,
            )
        )

    bdesc.wait()
    bvec = bias_v[...].at[jnp.zeros((L,), jnp.int32)].get(
        mode="promise_in_bounds")

    bounds = [0, 7, 14, 20, F]
    for g in range(len(bounds) - 1):
        lo, hi = bounds[g], bounds[g + 1]
        for d in gdescs[lo:hi]:
            d.wait()

        def _red(j, _, lo=lo, hi=hi, first=(g == 0)):
            o = j * L
            a = bvec + vals_v[pl.ds(o, L)] if first else acc_v[pl.ds(o, L)]
            for f in range(1 if first else lo, hi):
                a = a + vals_v[pl.ds(f * BPW + o, L)]
            acc_v[pl.ds(o, L)] = a
            return _

        lax.fori_loop(0, OUTV, _red, 0)
    pltpu.sync_copy(acc_v, out_hbm.at[0, pl.ds(base, BPW)])


def kernel(x, table, bias):
    out = _fl_kernel(x.T, table.reshape(1, TOTAL), bias)
    return out.reshape(B, 1)


# speedup vs baseline: 1.0056x; 1.0056x over previous
import functools

import jax
import jax.numpy as jnp
from jax import lax
from jax.experimental import pallas as pl
from jax.experimental.pallas import tpu as pltpu
from jax.experimental.pallas import tpu_sc as plsc

F = 26
FIELD = 100000
B = 16384
NC, NS, L = 2, 16, 16
NW = NC * NS
BPW = B // NW
E = BPW * F
OUTV = BPW // L
TOTAL = F * FIELD

_mesh = plsc.VectorSubcoreMesh(core_axis_name="c", subcore_axis_name="s")


@functools.partial(
    pl.kernel,
    out_type=jax.ShapeDtypeStruct((1, B), jnp.float32),
    mesh=_mesh,
    scratch_types=[
        pltpu.VMEM((E,), jnp.int32),
        pltpu.VMEM((E,), jnp.float32),
        pltpu.VMEM((BPW,), jnp.float32),
        pltpu.VMEM((L,), jnp.float32),
        pltpu.SemaphoreType.DMA,
        pltpu.SemaphoreType.DMA,
    ],
)
def _fl_kernel(xt_hbm, table_hbm, bias_hbm, out_hbm, idx_v, vals_v, acc_v,
               bias_v, semx, semg):
    wid = lax.axis_index("s") * NC + lax.axis_index("c")
    base = wid * BPW
    flat = table_hbm.at[0]

    xdescs = [
        pltpu.async_copy(
            xt_hbm.at[f, pl.ds(base, BPW)], idx_v.at[pl.ds(f * BPW, BPW)], semx
        )
        for f in range(F)
    ]
    bdesc = pltpu.async_copy(bias_hbm, bias_v.at[pl.ds(0, 1)], semx)
    gdescs = []
    for f in range(F):
        xdescs[f].wait()
        gdescs.append(
            pltpu.async_copy(
                flat.at[pl.ds(f * FIELD, FIELD)].at[idx_v.at[pl.ds(f * BPW, BPW)]],
                vals_v.at[pl.ds(f * BPW, BPW)],
                semg,
            )
        )

    bdesc.wait()
    bvec = bias_v[...].at[jnp.zeros((L,), jnp.int32)].get(
        mode="promise_in_bounds")
    HALF = F // 2

    for d in gdescs[:HALF]:
        d.wait()

    def _red_lo(j, _):
        o = j * L
        a = bvec + vals_v[pl.ds(o, L)]
        for f in range(1, HALF):
            a = a + vals_v[pl.ds(f * BPW + o, L)]
        acc_v[pl.ds(o, L)] = a
        return _

    lax.fori_loop(0, OUTV, _red_lo, 0)

    for d in gdescs[HALF:]:
        d.wait()

    def _red_hi(j, _):
        o = j * L
        a = acc_v[pl.ds(o, L)]
        for f in range(HALF, F):
            a = a + vals_v[pl.ds(f * BPW + o, L)]
        acc_v[pl.ds(o, L)] = a
        return _

    lax.fori_loop(0, OUTV, _red_hi, 0)
    pltpu.sync_copy(acc_v, out_hbm.at[0, pl.ds(base, BPW)])


def kernel(x, table, bias):
    out = _fl_kernel(x.T, table.reshape(1, TOTAL), bias)
    return out.reshape(B, 1)
